# baseline (speedup 1.0000x reference)
import functools

import jax
import jax.numpy as jnp
import numpy as np
from jax import lax
from jax.experimental import pallas as pl
from jax.experimental.pallas import tpu as pltpu
from jax.experimental.pallas import tpu_sc as plsc

B = 16384
L = 200
EMB = 64
H1 = 512
H2 = 256
OUT = 128
VOCAB = 1000000
W32 = EMB // 2

NC = 2
NS = 16
NW = NC * NS
SEQ_PER_W = B // NW
SBLK = 16
NSB = SEQ_PER_W // SBLK
HALF = L // 2
NH = 2 * SBLK
LANES = 16
RING = 4

CB = 256
NCB = VOCAB // CB
TAIL = VOCAB - NCB * CB
K1MAX = NCB + 2
K1OUTER = (K1MAX + 2 * NW - 1) // (2 * NW)

PERM = np.concatenate([
    np.arange(0, 32, 2), np.arange(1, 32, 2),
    np.arange(32, 64, 2), np.arange(33, 64, 2),
])

_SC_LINEAR = pltpu.CompilerParams(
    use_tc_tiling_on_sc=False, needs_layout_passes=False)
_SC_TCTILED = pltpu.CompilerParams(
    use_tc_tiling_on_sc=True, needs_layout_passes=False)


def _mesh():
    return plsc.VectorSubcoreMesh(
        core_axis_name="c", subcore_axis_name="s",
        num_cores=NC, num_subcores=NS)


def _make_pack_transpose():
    @functools.partial(
        pl.kernel,
        out_type=jax.ShapeDtypeStruct((VOCAB * W32,), jnp.int32),
        mesh=_mesh(),
        scratch_types=[
            pltpu.VMEM((EMB, CB), jnp.float32),
            pltpu.VMEM((EMB, CB), jnp.float32),
            pltpu.VMEM((CB * W32,), jnp.int32),
            pltpu.VMEM((CB * W32,), jnp.int32),
            pltpu.VMEM((TAIL * W32,), jnp.int32),
            pltpu.SemaphoreType.DMA,
            pltpu.SemaphoreType.DMA,
            pltpu.SemaphoreType.DMA,
            pltpu.SemaphoreType.DMA,
        ],
        compiler_params=_SC_TCTILED,
    )
    def k1(tt_hbm, tail_hbm, z_hbm, in0, in1, ob0, ob1, tv,
           is0, is1, os0, os1):
        wid = lax.axis_index("s") * NC + lax.axis_index("c")
        ins = (in0, in1)
        obs = (ob0, ob1)
        isems = (is0, is1)
        osems = (os0, os1)
        iota = lax.iota(jnp.int32, LANES)
        row_a = iota * 2
        row_b = iota * 2 + 1

        def jof(k):
            return wid + k * NW

        def stage(k, ib, isem):
            j = jof(k)

            @pl.when(j < NCB)
            def _():
                va = j * CB
                for r in range(EMB // 8):
                    pltpu.async_copy(
                        tt_hbm.at[pl.ds(r * 8, 8), pl.ds(va, CB)],
                        ib.at[pl.ds(r * 8, 8), :], isem)

        def process(k, ib, ob, isem, osem):
            j = jof(k)
            jprev = jof(k - 2)

            @pl.when((jprev >= 0) & (jprev < NCB))
            def _():
                pltpu.make_async_copy(
                    ob, z_hbm.at[pl.ds(0, CB * W32)], osem).wait()

            @pl.when(j < NCB)
            def _():
                for _r in range(EMB // 8):
                    pltpu.make_async_copy(
                        tt_hbm.at[pl.ds(0, 8), pl.ds(0, CB)],
                        ib.at[pl.ds(0, 8), :], isem).wait()

                def rne16(f):
                    w = plsc.bitcast(f, jnp.int32)
                    lsb = lax.shift_right_logical(w, 16) & 1
                    return lax.shift_right_logical(w + 0x7FFF + lsb, 16)

                @plsc.parallel_loop(0, CB, 1, unroll=4)
                def col(c):
                    cc = jnp.zeros((LANES,), jnp.int32) + c
                    for half in range(2):
                        wa = plsc.load_gather(ib, [row_a + half * 32, cc])
                        wb = plsc.load_gather(ib, [row_b + half * 32, cc])
                        p = rne16(wa) | lax.shift_left(rne16(wb), 16)
                        ob[pl.ds(c * W32 + half * LANES, LANES)] = p
                pltpu.async_copy(
                    ob, z_hbm.at[pl.ds(j * CB * W32, CB * W32)], osem)

        stage(0, in0, is0)

        def body(kk, carry):
            ka = 2 * kk
            stage(ka + 1, in1, is1)
            process(ka, in0, ob0, is0, os0)
            stage(ka + 2, in0, is0)
            process(ka + 1, in1, ob1, is1, os1)
            return carry

        lax.fori_loop(0, K1OUTER, body, 0)

        for kd, ob, osem in ((2 * K1OUTER - 2, ob0, os0),
                             (2 * K1OUTER - 1, ob1, os1)):
            @pl.when(jof(kd) < NCB)
            def _():
                pltpu.make_async_copy(
                    ob, z_hbm.at[pl.ds(0, CB * W32)], osem).wait()

        @pl.when(wid == 0)
        def _():
            pltpu.sync_copy(tail_hbm, tv)
            pltpu.sync_copy(tv, z_hbm.at[pl.ds(NCB * CB * W32, TAIL * W32)])

    return k1


def _accum(rows_ref, acc):
    def body(i, acc):
        a = list(acc)
        for k in range(2):
            r = 2 * i + k
            w0 = plsc.bitcast(rows_ref[r, pl.ds(0, LANES)], jnp.bfloat16)
            w1 = plsc.bitcast(rows_ref[r, pl.ds(LANES, LANES)], jnp.bfloat16)
            e0, o0 = plsc.unpack(w0, format=plsc.PackFormat.INTERLEAVED,
                                 preferred_element_type=jnp.float32)
            e1, o1 = plsc.unpack(w1, format=plsc.PackFormat.INTERLEAVED,
                                 preferred_element_type=jnp.float32)
            a[4 * k + 0] += e0
            a[4 * k + 1] += o0
            a[4 * k + 2] += e1
            a[4 * k + 3] += o1
        return tuple(a)

    return lax.fori_loop(0, HALF // 2, body, acc)


def _make_pool():
    @functools.partial(
        pl.kernel,
        out_type=jax.ShapeDtypeStruct((B * EMB,), jnp.float32),
        mesh=_mesh(),
        scratch_types=[
            pltpu.VMEM((NH, HALF), jnp.int32),
            pltpu.VMEM((NH, HALF), jnp.int32),
            pltpu.VMEM((HALF, W32), jnp.int32),
            pltpu.VMEM((HALF, W32), jnp.int32),
            pltpu.VMEM((HALF, W32), jnp.int32),
            pltpu.VMEM((HALF, W32), jnp.int32),
            pltpu.VMEM((SBLK * EMB,), jnp.float32),
            pltpu.VMEM((SBLK * EMB,), jnp.float32),
            pltpu.SemaphoreType.DMA,
            pltpu.SemaphoreType.DMA,
            pltpu.SemaphoreType.DMA,
            pltpu.SemaphoreType.DMA,
            pltpu.SemaphoreType.DMA,
            pltpu.SemaphoreType.DMA,
            pltpu.SemaphoreType.DMA,
            pltpu.SemaphoreType.DMA,
        ],
        compiler_params=_SC_LINEAR,
    )
    def pool(x_hbm, table_hbm, out_hbm, idx0, idx1, r0, r1, r2, r3,
             ov0, ov1, isem0, isem1, rs0, rs1, rs2, rs3, osem0, osem1):
        wid = lax.axis_index("s") * NC + lax.axis_index("c")
        seq_base = wid * SEQ_PER_W
        rows = (r0, r1, r2, r3)
        rsems = (rs0, rs1, rs2, rs3)

        def idx_copy(sb, ib, isem):
            s0 = seq_base + sb * SBLK
            return pltpu.async_copy(x_hbm.at[pl.ds(s0 * 2, NH)], ib, isem)

        def gather(ib, h, ring_pos):
            return pltpu.async_copy(
                table_hbm.at[ib.at[h]], rows[ring_pos], rsems[ring_pos])

        def process(sb, ib, ov, osem, k):
            @pl.when(k > 0)
            def _():
                pltpu.make_async_copy(
                    ov, out_hbm.at[pl.ds(0, SBLK * EMB)], osem).wait()

            for h in range(3):
                gather(ib, h, h)
            acc = None
            for h in range(NH):
                if h + 3 < NH:
                    gather(ib, h + 3, (h + 3) % RING)
                pltpu.make_async_copy(
                    table_hbm.at[ib.at[h]], rows[h % RING],
                    rsems[h % RING]).wait()
                if h % 2 == 0:
                    acc = tuple(jnp.zeros((LANES,), jnp.float32)
                                for _ in range(8))
                acc = _accum(rows[h % RING], acc)
                if h % 2 == 1:
                    s = h // 2
                    for c in range(4):
                        ov[pl.ds(s * EMB + c * LANES, LANES)] = (
                            (acc[c] + acc[4 + c]) * (1.0 / L))
            s0 = seq_base + sb * SBLK
            pltpu.async_copy(ov, out_hbm.at[pl.ds(s0 * EMB, SBLK * EMB)],
                             osem)

        idx_copy(0, idx0, isem0)

        def body(k, carry):
            sa = 2 * k
            pltpu.make_async_copy(
                x_hbm.at[pl.ds(0, NH)], idx0, isem0).wait()
            idx_copy(sa + 1, idx1, isem1)
            process(sa, idx0, ov0, osem0, k)
            pltpu.make_async_copy(
                x_hbm.at[pl.ds(0, NH)], idx1, isem1).wait()

            @pl.when(k + 1 < NSB // 2)
            def _():
                idx_copy(sa + 2, idx0, isem0)

            process(sa + 1, idx1, ov1, osem1, k)
            return carry

        lax.fori_loop(0, NSB // 2, body, 0)
        pltpu.make_async_copy(
            ov0, out_hbm.at[pl.ds(0, SBLK * EMB)], osem0).wait()
        pltpu.make_async_copy(
            ov1, out_hbm.at[pl.ds(0, SBLK * EMB)], osem1).wait()

    return pool


_pack_transpose = _make_pack_transpose()
_pool = _make_pool()


def _mlp(pooled, W1, b1, W2, b2, W3, b3):
    BM = 2048

    def body(x_ref, w1, b1r, w2, b2r, w3, b3r, o_ref):
        h = jnp.dot(x_ref[...], w1[...],
                    preferred_element_type=jnp.float32) + b1r[...]
        h = jnp.maximum(h, 0.0)
        h = jnp.dot(h, w2[...], preferred_element_type=jnp.float32) + b2r[...]
        h = jnp.maximum(h, 0.0)
        o_ref[...] = jnp.dot(h, w3[...],
                             preferred_element_type=jnp.float32) + b3r[...]

    return pl.pallas_call(
        body,
        grid=(B // BM,),
        in_specs=[
            pl.BlockSpec((BM, EMB), lambda i: (i, 0)),
            pl.BlockSpec((EMB, H1), lambda i: (0, 0)),
            pl.BlockSpec((1, H1), lambda i: (0, 0)),
            pl.BlockSpec((H1, H2), lambda i: (0, 0)),
            pl.BlockSpec((1, H2), lambda i: (0, 0)),
            pl.BlockSpec((H2, OUT), lambda i: (0, 0)),
            pl.BlockSpec((1, OUT), lambda i: (0, 0)),
        ],
        out_specs=pl.BlockSpec((BM, OUT), lambda i: (i, 0)),
        out_shape=jax.ShapeDtypeStruct((B, OUT), jnp.float32),
    )(pooled, W1, b1, W2, b2, W3, b3)


def kernel(x, table, W1, b1, W2, b2, W3, b3):
    x2 = x.reshape(2 * B, HALF)
    tail = lax.bitcast_convert_type(
        table[NCB * CB:].astype(jnp.bfloat16).reshape(TAIL, W32, 2),
        jnp.int32).reshape(-1)
    z = _pack_transpose(table.T, tail)
    pooled = _pool(x2, z.reshape(VOCAB, W32)).reshape(B, EMB)
    W1p = W1[PERM]
    return _mlp(pooled, W1p, b1.reshape(1, H1), W2, b2.reshape(1, H2),
                W3, b3.reshape(1, OUT))


# speedup vs baseline: 1.1590x; 1.1590x over previous
import functools

import jax
import jax.numpy as jnp
import numpy as np
from jax import lax
from jax.experimental import pallas as pl
from jax.experimental.pallas import tpu as pltpu
from jax.experimental.pallas import tpu_sc as plsc

B = 16384
L = 200
EMB = 64
H1 = 512
H2 = 256
OUT = 128
VOCAB = 1000000
W32 = EMB // 2

NC = 2
NS = 16
NW = NC * NS
SEQ_PER_W = B // NW
SBLK = 16
NSB = SEQ_PER_W // SBLK
HALF = L // 2
NH = 2 * SBLK
LANES = 16
RING = 4

CB = 256
NCB = VOCAB // CB
TAIL = VOCAB - NCB * CB
K1MAX = NCB + 2
K1OUTER = (K1MAX + 2 * NW - 1) // (2 * NW)

PERM = np.concatenate([
    np.arange(0, 16), np.arange(32, 48),
    np.arange(16, 32), np.arange(48, 64),
])
CBP = CB + 1

_SC_LINEAR = pltpu.CompilerParams(
    use_tc_tiling_on_sc=False, needs_layout_passes=False)
_SC_TCTILED = pltpu.CompilerParams(
    use_tc_tiling_on_sc=True, needs_layout_passes=False)


def _mesh():
    return plsc.VectorSubcoreMesh(
        core_axis_name="c", subcore_axis_name="s",
        num_cores=NC, num_subcores=NS)


def _make_pack_transpose():
    @functools.partial(
        pl.kernel,
        out_type=jax.ShapeDtypeStruct((VOCAB * W32,), jnp.int32),
        mesh=_mesh(),
        scratch_types=[
            pltpu.VMEM((EMB, CBP), jnp.float32),
            pltpu.VMEM((EMB, CBP), jnp.float32),
            pltpu.VMEM((CB * W32,), jnp.int32),
            pltpu.VMEM((CB * W32,), jnp.int32),
            pltpu.VMEM((TAIL * W32,), jnp.int32),
            pltpu.SemaphoreType.DMA,
            pltpu.SemaphoreType.DMA,
            pltpu.SemaphoreType.DMA,
            pltpu.SemaphoreType.DMA,
        ],
        compiler_params=_SC_TCTILED,
    )
    def k1(tt_hbm, tail_hbm, z_hbm, in0, in1, ob0, ob1, tv,
           is0, is1, os0, os1):
        wid = lax.axis_index("s") * NC + lax.axis_index("c")
        ins = (in0, in1)
        obs = (ob0, ob1)
        isems = (is0, is1)
        osems = (os0, os1)
        iota = lax.iota(jnp.int32, LANES)

        def jof(k):
            return wid + k * NW

        def stage(k, ib, isem):
            j = jof(k)

            @pl.when(j < NCB)
            def _():
                va = j * CB
                for r in range(EMB // 8):
                    pltpu.async_copy(
                        tt_hbm.at[pl.ds(r * 8, 8), pl.ds(va, CB)],
                        ib.at[pl.ds(r * 8, 8), pl.ds(0, CB)], isem)

        def process(k, ib, ob, isem, osem):
            j = jof(k)
            jprev = jof(k - 2)

            @pl.when((jprev >= 0) & (jprev < NCB))
            def _():
                pltpu.make_async_copy(
                    ob, z_hbm.at[pl.ds(0, CB * W32)], osem).wait()

            @pl.when(j < NCB)
            def _():
                for _r in range(EMB // 8):
                    pltpu.make_async_copy(
                        tt_hbm.at[pl.ds(0, 8), pl.ds(0, CB)],
                        ib.at[pl.ds(0, 8), pl.ds(0, CB)], isem).wait()

                @plsc.parallel_loop(0, CB, 1, unroll=4)
                def col(c):
                    cc = jnp.zeros((LANES,), jnp.int32) + c
                    for half in range(2):
                        wa = plsc.load_gather(ib, [iota + half * 16, cc])
                        wb = plsc.load_gather(ib, [iota + half * 16 + 32, cc])
                        p = plsc.pack(wa, wb,
                                      format=plsc.PackFormat.INTERLEAVED)
                        ob[pl.ds(c * W32 + half * LANES, LANES)] = (
                            plsc.bitcast(p, jnp.int32))
                pltpu.async_copy(
                    ob, z_hbm.at[pl.ds(j * CB * W32, CB * W32)], osem)

        stage(0, in0, is0)

        def body(kk, carry):
            ka = 2 * kk
            stage(ka + 1, in1, is1)
            process(ka, in0, ob0, is0, os0)
            stage(ka + 2, in0, is0)
            process(ka + 1, in1, ob1, is1, os1)
            return carry

        lax.fori_loop(0, K1OUTER, body, 0)

        for kd, ob, osem in ((2 * K1OUTER - 2, ob0, os0),
                             (2 * K1OUTER - 1, ob1, os1)):
            @pl.when(jof(kd) < NCB)
            def _():
                pltpu.make_async_copy(
                    ob, z_hbm.at[pl.ds(0, CB * W32)], osem).wait()

        @pl.when(wid == 0)
        def _():
            pltpu.sync_copy(tail_hbm, tv)
            pltpu.sync_copy(tv, z_hbm.at[pl.ds(NCB * CB * W32, TAIL * W32)])

    return k1


def _accum(rows_ref, acc):
    def body(i, acc):
        a = list(acc)
        for k in range(2):
            r = 2 * i + k
            w0 = plsc.bitcast(rows_ref[r, pl.ds(0, LANES)], jnp.bfloat16)
            w1 = plsc.bitcast(rows_ref[r, pl.ds(LANES, LANES)], jnp.bfloat16)
            e0, o0 = plsc.unpack(w0, format=plsc.PackFormat.INTERLEAVED,
                                 preferred_element_type=jnp.float32)
            e1, o1 = plsc.unpack(w1, format=plsc.PackFormat.INTERLEAVED,
                                 preferred_element_type=jnp.float32)
            a[4 * k + 0] += e0
            a[4 * k + 1] += o0
            a[4 * k + 2] += e1
            a[4 * k + 3] += o1
        return tuple(a)

    return lax.fori_loop(0, HALF // 2, body, acc)


def _make_pool():
    @functools.partial(
        pl.kernel,
        out_type=jax.ShapeDtypeStruct((B * EMB,), jnp.float32),
        mesh=_mesh(),
        scratch_types=[
            pltpu.VMEM((NH, HALF), jnp.int32),
            pltpu.VMEM((NH, HALF), jnp.int32),
            pltpu.VMEM((HALF, W32), jnp.int32),
            pltpu.VMEM((HALF, W32), jnp.int32),
            pltpu.VMEM((HALF, W32), jnp.int32),
            pltpu.VMEM((HALF, W32), jnp.int32),
            pltpu.VMEM((SBLK * EMB,), jnp.float32),
            pltpu.VMEM((SBLK * EMB,), jnp.float32),
            pltpu.SemaphoreType.DMA,
            pltpu.SemaphoreType.DMA,
            pltpu.SemaphoreType.DMA,
            pltpu.SemaphoreType.DMA,
            pltpu.SemaphoreType.DMA,
            pltpu.SemaphoreType.DMA,
            pltpu.SemaphoreType.DMA,
            pltpu.SemaphoreType.DMA,
        ],
        compiler_params=_SC_LINEAR,
    )
    def pool(x_hbm, table_hbm, out_hbm, idx0, idx1, r0, r1, r2, r3,
             ov0, ov1, isem0, isem1, rs0, rs1, rs2, rs3, osem0, osem1):
        wid = lax.axis_index("s") * NC + lax.axis_index("c")
        seq_base = wid * SEQ_PER_W
        rows = (r0, r1, r2, r3)
        rsems = (rs0, rs1, rs2, rs3)

        def idx_copy(sb, ib, isem):
            s0 = seq_base + sb * SBLK
            return pltpu.async_copy(x_hbm.at[pl.ds(s0 * 2, NH)], ib, isem)

        def gather(ib, h, ring_pos):
            return pltpu.async_copy(
                table_hbm.at[ib.at[h]], rows[ring_pos], rsems[ring_pos])

        def process(sb, ib, ov, osem, k):
            @pl.when(k > 0)
            def _():
                pltpu.make_async_copy(
                    ov, out_hbm.at[pl.ds(0, SBLK * EMB)], osem).wait()

            for h in range(3):
                gather(ib, h, h)
            acc = None
            for h in range(NH):
                if h + 3 < NH:
                    gather(ib, h + 3, (h + 3) % RING)
                pltpu.make_async_copy(
                    table_hbm.at[ib.at[h]], rows[h % RING],
                    rsems[h % RING]).wait()
                if h % 2 == 0:
                    acc = tuple(jnp.zeros((LANES,), jnp.float32)
                                for _ in range(8))
                acc = _accum(rows[h % RING], acc)
                if h % 2 == 1:
                    s = h // 2
                    for c in range(4):
                        ov[pl.ds(s * EMB + c * LANES, LANES)] = (
                            (acc[c] + acc[4 + c]) * (1.0 / L))
            s0 = seq_base + sb * SBLK
            pltpu.async_copy(ov, out_hbm.at[pl.ds(s0 * EMB, SBLK * EMB)],
                             osem)

        idx_copy(0, idx0, isem0)

        def body(k, carry):
            sa = 2 * k
            pltpu.make_async_copy(
                x_hbm.at[pl.ds(0, NH)], idx0, isem0).wait()
            idx_copy(sa + 1, idx1, isem1)
            process(sa, idx0, ov0, osem0, k)
            pltpu.make_async_copy(
                x_hbm.at[pl.ds(0, NH)], idx1, isem1).wait()

            @pl.when(k + 1 < NSB // 2)
            def _():
                idx_copy(sa + 2, idx0, isem0)

            process(sa + 1, idx1, ov1, osem1, k)
            return carry

        lax.fori_loop(0, NSB // 2, body, 0)
        pltpu.make_async_copy(
            ov0, out_hbm.at[pl.ds(0, SBLK * EMB)], osem0).wait()
        pltpu.make_async_copy(
            ov1, out_hbm.at[pl.ds(0, SBLK * EMB)], osem1).wait()

    return pool


_pack_transpose = _make_pack_transpose()
_pool = _make_pool()


def _mlp(pooled, W1, b1, W2, b2, W3, b3):
    BM = 2048

    def body(x_ref, w1, b1r, w2, b2r, w3, b3r, o_ref):
        h = jnp.dot(x_ref[...], w1[...],
                    preferred_element_type=jnp.float32) + b1r[...]
        h = jnp.maximum(h, 0.0)
        h = jnp.dot(h, w2[...], preferred_element_type=jnp.float32) + b2r[...]
        h = jnp.maximum(h, 0.0)
        o_ref[...] = jnp.dot(h, w3[...],
                             preferred_element_type=jnp.float32) + b3r[...]

    return pl.pallas_call(
        body,
        grid=(B // BM,),
        in_specs=[
            pl.BlockSpec((BM, EMB), lambda i: (i, 0)),
            pl.BlockSpec((EMB, H1), lambda i: (0, 0)),
            pl.BlockSpec((1, H1), lambda i: (0, 0)),
            pl.BlockSpec((H1, H2), lambda i: (0, 0)),
            pl.BlockSpec((1, H2), lambda i: (0, 0)),
            pl.BlockSpec((H2, OUT), lambda i: (0, 0)),
            pl.BlockSpec((1, OUT), lambda i: (0, 0)),
        ],
        out_specs=pl.BlockSpec((BM, OUT), lambda i: (i, 0)),
        out_shape=jax.ShapeDtypeStruct((B, OUT), jnp.float32),
    )(pooled, W1, b1, W2, b2, W3, b3)


def kernel(x, table, W1, b1, W2, b2, W3, b3):
    x2 = x.reshape(2 * B, HALF)
    tail = lax.bitcast_convert_type(
        table[NCB * CB:].astype(jnp.bfloat16).reshape(TAIL, W32, 2),
        jnp.int32).reshape(-1)
    z = _pack_transpose(table.T, tail)
    pooled = _pool(x2, z.reshape(VOCAB, W32)).reshape(B, EMB)
    W1p = W1[PERM]
    return _mlp(pooled, W1p, b1.reshape(1, H1), W2, b2.reshape(1, H2),
                W3, b3.reshape(1, OUT))


# speedup vs baseline: 1.3966x; 1.2051x over previous
import functools

import jax
import jax.numpy as jnp
from jax import lax
from jax.experimental import pallas as pl
from jax.experimental.pallas import tpu as pltpu
from jax.experimental.pallas import tpu_sc as plsc

B = 16384
L = 200
EMB = 64
H1 = 512
H2 = 256
OUT = 128
VOCAB = 1000000

NC = 2
NS = 16
NW = NC * NS
SEQ_PER_W = B // NW
SBLK = 16
NSB = SEQ_PER_W // SBLK
HALF = L // 2
NH = 2 * SBLK
LANES = 16
RING = 4

_SC_PARAMS = pltpu.CompilerParams(
    use_tc_tiling_on_sc=False, needs_layout_passes=False)


def _accum(rows_ref, acc):
    def body(i, acc):
        a = list(acc)
        for k in range(2):
            r = 2 * i + k
            for c in range(4):
                a[4 * k + c] += rows_ref[r, pl.ds(c * LANES, LANES)]
        return tuple(a)

    return lax.fori_loop(0, HALF // 2, body, acc)


def _make_pool():
    mesh = plsc.VectorSubcoreMesh(
        core_axis_name="c", subcore_axis_name="s",
        num_cores=NC, num_subcores=NS)

    @functools.partial(
        pl.kernel,
        out_type=jax.ShapeDtypeStruct((B * EMB,), jnp.float32),
        mesh=mesh,
        scratch_types=[
            pltpu.VMEM((NH, HALF), jnp.int32),
            pltpu.VMEM((NH, HALF), jnp.int32),
            pltpu.VMEM((HALF, EMB), jnp.float32),
            pltpu.VMEM((HALF, EMB), jnp.float32),
            pltpu.VMEM((HALF, EMB), jnp.float32),
            pltpu.VMEM((HALF, EMB), jnp.float32),
            pltpu.VMEM((SBLK * EMB,), jnp.float32),
            pltpu.VMEM((SBLK * EMB,), jnp.float32),
            pltpu.SemaphoreType.DMA,
            pltpu.SemaphoreType.DMA,
            pltpu.SemaphoreType.DMA,
            pltpu.SemaphoreType.DMA,
            pltpu.SemaphoreType.DMA,
            pltpu.SemaphoreType.DMA,
            pltpu.SemaphoreType.DMA,
            pltpu.SemaphoreType.DMA,
        ],
        compiler_params=_SC_PARAMS,
    )
    def pool(x_hbm, table_hbm, out_hbm, idx0, idx1, r0, r1, r2, r3,
             ov0, ov1, isem0, isem1, rs0, rs1, rs2, rs3, osem0, osem1):
        wid = lax.axis_index("s") * NC + lax.axis_index("c")
        seq_base = wid * SEQ_PER_W
        rows = (r0, r1, r2, r3)
        rsems = (rs0, rs1, rs2, rs3)

        def idx_copy(sb, ib, isem):
            s0 = seq_base + sb * SBLK
            return pltpu.async_copy(x_hbm.at[pl.ds(s0 * 2, NH)], ib, isem)

        def gather(ib, h, ring_pos):
            return pltpu.async_copy(
                table_hbm.at[ib.at[h]], rows[ring_pos], rsems[ring_pos])

        def process(sb, ib, ov, osem, k):
            @pl.when(k > 0)
            def _():
                pltpu.make_async_copy(
                    ov, out_hbm.at[pl.ds(0, SBLK * EMB)], osem).wait()

            for h in range(3):
                gather(ib, h, h)
            acc = None
            for h in range(NH):
                if h + 3 < NH:
                    gather(ib, h + 3, (h + 3) % RING)
                pltpu.make_async_copy(
                    table_hbm.at[ib.at[h]], rows[h % RING],
                    rsems[h % RING]).wait()
                if h % 2 == 0:
                    acc = tuple(jnp.zeros((LANES,), jnp.float32)
                                for _ in range(8))
                acc = _accum(rows[h % RING], acc)
                if h % 2 == 1:
                    s = h // 2
                    for c in range(4):
                        ov[pl.ds(s * EMB + c * LANES, LANES)] = (
                            (acc[c] + acc[4 + c]) * (1.0 / L))
            s0 = seq_base + sb * SBLK
            pltpu.async_copy(ov, out_hbm.at[pl.ds(s0 * EMB, SBLK * EMB)],
                             osem)

        idx_copy(0, idx0, isem0)

        def body(k, carry):
            sa = 2 * k
            pltpu.make_async_copy(
                x_hbm.at[pl.ds(0, NH)], idx0, isem0).wait()
            idx_copy(sa + 1, idx1, isem1)
            process(sa, idx0, ov0, osem0, k)
            pltpu.make_async_copy(
                x_hbm.at[pl.ds(0, NH)], idx1, isem1).wait()

            @pl.when(k + 1 < NSB // 2)
            def _():
                idx_copy(sa + 2, idx0, isem0)

            process(sa + 1, idx1, ov1, osem1, k)
            return carry

        lax.fori_loop(0, NSB // 2, body, 0)
        pltpu.make_async_copy(
            ov0, out_hbm.at[pl.ds(0, SBLK * EMB)], osem0).wait()
        pltpu.make_async_copy(
            ov1, out_hbm.at[pl.ds(0, SBLK * EMB)], osem1).wait()

    return pool


_pool = _make_pool()


def _mlp(pooled, W1, b1, W2, b2, W3, b3):
    BM = 2048

    def body(x_ref, w1, b1r, w2, b2r, w3, b3r, o_ref):
        h = jnp.dot(x_ref[...], w1[...],
                    preferred_element_type=jnp.float32) + b1r[...]
        h = jnp.maximum(h, 0.0)
        h = jnp.dot(h, w2[...], preferred_element_type=jnp.float32) + b2r[...]
        h = jnp.maximum(h, 0.0)
        o_ref[...] = jnp.dot(h, w3[...],
                             preferred_element_type=jnp.float32) + b3r[...]

    return pl.pallas_call(
        body,
        grid=(B // BM,),
        in_specs=[
            pl.BlockSpec((BM, EMB), lambda i: (i, 0)),
            pl.BlockSpec((EMB, H1), lambda i: (0, 0)),
            pl.BlockSpec((1, H1), lambda i: (0, 0)),
            pl.BlockSpec((H1, H2), lambda i: (0, 0)),
            pl.BlockSpec((1, H2), lambda i: (0, 0)),
            pl.BlockSpec((H2, OUT), lambda i: (0, 0)),
            pl.BlockSpec((1, OUT), lambda i: (0, 0)),
        ],
        out_specs=pl.BlockSpec((BM, OUT), lambda i: (i, 0)),
        out_shape=jax.ShapeDtypeStruct((B, OUT), jnp.float32),
    )(pooled, W1, b1, W2, b2, W3, b3)


def kernel(x, table, W1, b1, W2, b2, W3, b3):
    x2 = x.reshape(2 * B, HALF)
    pooled = _pool(x2, table).reshape(B, EMB)
    return _mlp(pooled, W1, b1.reshape(1, H1), W2, b2.reshape(1, H2),
                W3, b3.reshape(1, OUT))
